# baseline (speedup 1.0000x reference)
import functools

import jax
import jax.numpy as jnp
from jax import lax
from jax.experimental import pallas as pl
from jax.experimental.pallas import tpu as pltpu
from jax.experimental.pallas import tpu_sc as plsc

NUM_WORKERS = 32
CHUNK = 1024
GW = 128


def kernel(input, w_q, w_scale, w_zero_point):
    B, H = input.shape
    V, D = w_q.shape
    N = B * H
    per_w = N // NUM_WORKERS
    n_chunks = per_w // CHUNK
    assert per_w * NUM_WORKERS == N and n_chunks * CHUNK == per_w

    idx = input.reshape(N // GW, GW).astype(jnp.int32)
    coef_a = (1.0 / w_scale).astype(jnp.float32)
    coef_b = (w_zero_point / w_scale).astype(jnp.float32)
    ab = jnp.stack([coef_a, coef_b])[:, None] * jnp.ones((2, 16), jnp.float32)

    mesh = plsc.VectorSubcoreMesh(core_axis_name="c", subcore_axis_name="s")

    @functools.partial(
        pl.kernel,
        out_type=jax.ShapeDtypeStruct((N, D), jnp.float32),
        mesh=mesh,
        compiler_params=pltpu.CompilerParams(use_tc_tiling_on_sc=False),
        scratch_types=[
            pltpu.VMEM((CHUNK // GW, GW), jnp.int32),
            pltpu.VMEM((CHUNK, D), jnp.float32),
            pltpu.VMEM((2, 16), jnp.float32),
            pltpu.SemaphoreType.DMA,
        ],
    )
    def sc_kernel(table_hbm, idx_hbm, ab_hbm, out_hbm, idx_v, rows_v, ab_v, sem):
        wid = lax.axis_index("s") * 2 + lax.axis_index("c")
        pltpu.sync_copy(ab_hbm, ab_v)
        va = ab_v[0, :]
        vb = ab_v[1, :]

        @pl.loop(0, n_chunks)
        def _(j):
            base = wid * per_w + j * CHUNK
            irow = wid * (per_w // GW) + j * (CHUNK // GW)
            pltpu.sync_copy(idx_hbm.at[pl.ds(irow, CHUNK // GW)], idx_v)
            copies = [
                pltpu.async_copy(
                    table_hbm.at[idx_v.at[t]],
                    rows_v.at[pl.ds(t * GW, GW)],
                    sem,
                )
                for t in range(CHUNK // GW)
            ]
            for cp in copies:
                cp.wait()

            @pl.loop(0, CHUNK)
            def _(r):
                for c in range(0, D, 16):
                    rows_v[r, pl.ds(c, 16)] = rows_v[r, pl.ds(c, 16)] * va + vb

            pltpu.sync_copy(rows_v, out_hbm.at[pl.ds(base, CHUNK)])

    out = sc_kernel(w_q, idx, ab)
    return out.reshape(B, H, D)


# speedup vs baseline: 1.3361x; 1.3361x over previous
import dataclasses
import functools

import jax
import jax.numpy as jnp
from jax import lax
from jax.experimental import pallas as pl
from jax.experimental.pallas import tpu as pltpu
from jax.experimental.pallas import tpu_sc as plsc

NUM_WORKERS = 32
GW = 128
BT_PER_W = 4


def _compiler_params():
    cp = pltpu.CompilerParams(use_tc_tiling_on_sc=False)
    if "needs_layout_passes" in pltpu.CompilerParams.__dataclass_fields__:
        cp = dataclasses.replace(cp, needs_layout_passes=False)
    return cp


def kernel(input, w_q, w_scale, w_zero_point):
    B, H = input.shape
    V, D = w_q.shape
    n_bt = B // GW
    n_dt = D // 8
    assert n_bt == NUM_WORKERS * BT_PER_W and D % 8 == 0 and B % GW == 0

    idx3 = input.T.astype(jnp.int32).reshape(H, n_bt, GW)
    coef_a = (1.0 / w_scale).astype(jnp.float32)
    coef_b = (w_zero_point / w_scale).astype(jnp.float32)
    ab = jnp.stack([coef_a, coef_b])[:, None] * jnp.ones((2, 16), jnp.float32)

    mesh = plsc.VectorSubcoreMesh(core_axis_name="c", subcore_axis_name="s")

    @functools.partial(
        pl.kernel,
        out_type=jax.ShapeDtypeStruct((H, n_dt, n_bt, 8, GW), jnp.float32),
        mesh=mesh,
        compiler_params=_compiler_params(),
        scratch_types=[
            pltpu.VMEM((BT_PER_W, GW), jnp.int32),
            pltpu.VMEM((BT_PER_W * GW, D), jnp.float32),
            pltpu.VMEM((n_dt, BT_PER_W, 8, GW), jnp.float32),
            pltpu.VMEM((2, 16), jnp.float32),
            pltpu.SemaphoreType.DMA,
        ],
    )
    def sc_kernel(table_hbm, idx_hbm, ab_hbm, out_hbm, idx_v, rows_v, tiles_v, ab_v, sem):
        wid = lax.axis_index("s") * 2 + lax.axis_index("c")
        bt0 = wid * BT_PER_W
        pltpu.sync_copy(ab_hbm, ab_v)
        va = ab_v[0, :]
        vb = ab_v[1, :]
        iota16 = lax.iota(jnp.int32, 16)

        @pl.loop(0, H)
        def _(h):
            pltpu.sync_copy(idx_hbm.at[h, pl.ds(bt0, BT_PER_W)], idx_v)
            copies = [
                pltpu.async_copy(
                    table_hbm.at[idx_v.at[k]],
                    rows_v.at[pl.ds(k * GW, GW)],
                    sem,
                )
                for k in range(BT_PER_W)
            ]
            for cp in copies:
                cp.wait()

            @pl.loop(0, BT_PER_W)
            def _(bk):
                rbase = bk * GW
                for bc0 in range(0, GW, 16):
                    row_ids = iota16 + (rbase + bc0)
                    for dt in range(n_dt):
                        for dr in range(8):
                            col_ids = jnp.full((16,), dt * 8 + dr, jnp.int32)
                            v = plsc.load_gather(rows_v, [row_ids, col_ids])
                            tiles_v[dt, bk, dr, pl.ds(bc0, 16)] = v * va + vb

            for dt in range(n_dt):
                pltpu.sync_copy(
                    tiles_v.at[dt],
                    out_hbm.at[h, dt, pl.ds(bt0, BT_PER_W)],
                )

    out5 = sc_kernel(w_q, idx3, ab)
    return out5.transpose((2, 4, 0, 1, 3)).reshape(B, H, D)


# speedup vs baseline: 1.4260x; 1.0673x over previous
import dataclasses
import functools

import numpy as np

import jax
import jax.numpy as jnp
from jax import lax
from jax.experimental import pallas as pl
from jax.experimental.pallas import tpu as pltpu
from jax.experimental.pallas import tpu_sc as plsc

NUM_WORKERS = 32
GW = 128
BT_PER_W = 4
L = 16


def _compiler_params():
    cp = pltpu.CompilerParams(use_tc_tiling_on_sc=False)
    if "needs_layout_passes" in pltpu.CompilerParams.__dataclass_fields__:
        cp = dataclasses.replace(cp, needs_layout_passes=False)
    return cp


def kernel(input, w_q, w_scale, w_zero_point):
    B, H = input.shape
    V, D = w_q.shape
    n_bt = B // GW
    n_dt = D // 8
    rows_w = BT_PER_W * GW
    tile_words = 8 * GW
    assert n_bt == NUM_WORKERS * BT_PER_W and D % 8 == 0 and B % GW == 0

    idx3 = input.T.astype(jnp.int32).reshape(H, n_bt, GW)
    coef_a = (1.0 / w_scale).astype(jnp.float32)
    coef_b = (w_zero_point / w_scale).astype(jnp.float32)
    ab = jnp.stack([coef_a, coef_b])[:, None] * jnp.ones((2, L), jnp.float32)

    ks = np.arange(L)
    col_pat = []
    dst_pat = []
    for c0 in range(D):
        cc = (c0 + ks) % D
        col_pat.append(cc.astype(np.int32))
        dst_pat.append(((cc >> 3) * (BT_PER_W * tile_words)
                        + (cc & 7) * GW + ks).astype(np.int32))
    lane_pat = jnp.asarray(np.stack(col_pat + dst_pat))

    mesh = plsc.VectorSubcoreMesh(core_axis_name="c", subcore_axis_name="s")

    @functools.partial(
        pl.kernel,
        out_type=jax.ShapeDtypeStruct((H, n_dt, n_bt * tile_words), jnp.float32),
        mesh=mesh,
        compiler_params=_compiler_params(),
        scratch_types=[
            pltpu.VMEM((BT_PER_W, GW), jnp.int32),
            pltpu.VMEM((rows_w, D), jnp.float32),
            pltpu.VMEM((n_dt * BT_PER_W * tile_words,), jnp.float32),
            pltpu.VMEM((2, L), jnp.float32),
            pltpu.VMEM((2 * D, L), jnp.int32),
            pltpu.SemaphoreType.DMA,
        ],
    )
    def sc_kernel(table_hbm, idx_hbm, ab_hbm, pat_hbm, out_hbm,
                  idx_v, rows_v, tiles_v, ab_v, pat_v, sem):
        wid = lax.axis_index("s") * 2 + lax.axis_index("c")
        bt0 = wid * BT_PER_W
        pltpu.sync_copy(ab_hbm, ab_v)
        pltpu.sync_copy(pat_hbm, pat_v)
        va = ab_v[0, :]
        vb = ab_v[1, :]
        iota16 = lax.iota(jnp.int32, L)

        @pl.loop(0, H)
        def _(h):
            pltpu.sync_copy(idx_hbm.at[h, pl.ds(bt0, BT_PER_W)], idx_v)
            copies = [
                pltpu.async_copy(
                    table_hbm.at[idx_v.at[k]],
                    rows_v.at[pl.ds(k * GW, GW)],
                    sem,
                )
                for k in range(BT_PER_W)
            ]
            for cp in copies:
                cp.wait()

            @pl.loop(0, rows_w // L)
            def _(b16):
                r0 = b16 * L
                row_ids = iota16 + r0
                sbase = (b16 // 8) * tile_words + (b16 % 8) * L
                for c0 in range(D):
                    v = plsc.load_gather(rows_v, [row_ids, pat_v[c0, :]])
                    dst = pat_v[D + c0, :] + sbase
                    plsc.store_scatter(tiles_v, [dst], v * va + vb)

            for dt in range(n_dt):
                pltpu.sync_copy(
                    tiles_v.at[pl.ds(dt * BT_PER_W * tile_words, BT_PER_W * tile_words)],
                    out_hbm.at[h, dt, pl.ds(bt0 * tile_words, BT_PER_W * tile_words)],
                )

    out5 = sc_kernel(w_q, idx3, ab, lane_pat)
    return (
        out5.reshape(H, n_dt, n_bt, 8, GW)
        .transpose((2, 4, 0, 1, 3))
        .reshape(B, H, D)
    )


# speedup vs baseline: 1.5104x; 1.0592x over previous
import dataclasses
import functools

import numpy as np

import jax
import jax.numpy as jnp
from jax import lax
from jax.experimental import pallas as pl
from jax.experimental.pallas import tpu as pltpu
from jax.experimental.pallas import tpu_sc as plsc

NUM_WORKERS = 32
GW = 128
BT_PER_W = 4
L = 16


def _compiler_params():
    cp = pltpu.CompilerParams(use_tc_tiling_on_sc=False)
    if "needs_layout_passes" in pltpu.CompilerParams.__dataclass_fields__:
        cp = dataclasses.replace(cp, needs_layout_passes=False)
    return cp


def kernel(input, w_q, w_scale, w_zero_point):
    B, H = input.shape
    V, D = w_q.shape
    n_bt = B // GW
    n_dt = D // 8
    rows_w = BT_PER_W * GW
    tile_words = 8 * GW
    assert n_bt == NUM_WORKERS * BT_PER_W and D % 8 == 0 and B % GW == 0

    idx3 = input.T.astype(jnp.int32).reshape(H, n_bt, GW)
    coef_a = (1.0 / w_scale).astype(jnp.float32)
    coef_b = (w_zero_point / w_scale).astype(jnp.float32)
    ab = jnp.stack([coef_a, coef_b])[:, None] * jnp.ones((2, L), jnp.float32)

    ks = np.arange(L)
    col_pat = []
    dst_pat = []
    for c0 in range(D):
        cc = (c0 + ks) % D
        col_pat.append(cc.astype(np.int32))
        dst_pat.append(((cc >> 3) * (BT_PER_W * tile_words)
                        + (cc & 7) * GW + ks).astype(np.int32))
    lane_pat = jnp.asarray(np.stack(col_pat + dst_pat))

    mesh = plsc.VectorSubcoreMesh(core_axis_name="c", subcore_axis_name="s")

    @functools.partial(
        pl.kernel,
        out_type=jax.ShapeDtypeStruct((H, n_dt, n_bt * tile_words), jnp.float32),
        mesh=mesh,
        compiler_params=_compiler_params(),
        scratch_types=[
            pltpu.VMEM((2, BT_PER_W, GW), jnp.int32),
            pltpu.VMEM((2 * rows_w, D), jnp.float32),
            pltpu.VMEM((2 * n_dt * BT_PER_W * tile_words,), jnp.float32),
            pltpu.VMEM((2, L), jnp.float32),
            pltpu.VMEM((2 * D, L), jnp.int32),
            pltpu.SemaphoreType.DMA,
            pltpu.SemaphoreType.DMA,
            pltpu.SemaphoreType.DMA,
        ],
    )
    def sc_kernel(table_hbm, idx_hbm, ab_hbm, pat_hbm, out_hbm,
                  idx_v, rows_v, tiles_v, ab_v, pat_v, sem_i, sem_g, sem_w):
        wid = lax.axis_index("s") * 2 + lax.axis_index("c")
        bt0 = wid * BT_PER_W
        pltpu.sync_copy(ab_hbm, ab_v)
        pltpu.sync_copy(pat_hbm, pat_v)
        va = ab_v[0, :]
        vb = ab_v[1, :]
        iota16 = lax.iota(jnp.int32, L)
        nt = n_dt * BT_PER_W * tile_words

        def compute(p):
            @pl.loop(0, rows_w // L)
            def _(b16):
                row_ids = iota16 + (p * rows_w + b16 * L)
                sbase = p * nt + (b16 // 8) * tile_words + (b16 % 8) * L
                for c0 in range(D):
                    v = plsc.load_gather(rows_v, [row_ids, pat_v[c0, :]])
                    dst = pat_v[D + c0, :] + sbase
                    plsc.store_scatter(tiles_v, [dst], v * va + vb)

        def gathers(p):
            return [
                pltpu.async_copy(
                    table_hbm.at[idx_v.at[p, k]],
                    rows_v.at[pl.ds(p * rows_w + k * GW, GW)],
                    sem_g,
                )
                for k in range(BT_PER_W)
            ]

        def writes(p, h):
            return [
                pltpu.async_copy(
                    tiles_v.at[pl.ds(p * nt + dt * BT_PER_W * tile_words,
                                     BT_PER_W * tile_words)],
                    out_hbm.at[h, dt, pl.ds(bt0 * tile_words,
                                            BT_PER_W * tile_words)],
                    sem_w,
                )
                for dt in range(n_dt)
            ]

        @pl.loop(0, H // 2)
        def _(j):
            h0 = 2 * j
            ci0 = pltpu.async_copy(idx_hbm.at[h0, pl.ds(bt0, BT_PER_W)],
                                   idx_v.at[0], sem_i)
            ci1 = pltpu.async_copy(idx_hbm.at[h0 + 1, pl.ds(bt0, BT_PER_W)],
                                   idx_v.at[1], sem_i)
            ci0.wait()
            g0 = gathers(0)
            ci1.wait()
            g1 = gathers(1)
            for cp in g0:
                cp.wait()
            compute(0)
            w0 = writes(0, h0)
            for cp in g1:
                cp.wait()
            compute(1)
            w1 = writes(1, h0 + 1)
            for cp in w0 + w1:
                cp.wait()

    out5 = sc_kernel(w_q, idx3, ab, lane_pat)
    return (
        out5.reshape(H, n_dt, n_bt, 8, GW)
        .transpose((2, 4, 0, 1, 3))
        .reshape(B, H, D)
    )
